# Initial kernel scaffold; baseline (speedup 1.0000x reference)
import jax
import jax.numpy as jnp
from jax.experimental import pallas as pl


def kernel(router_input, x, Wr, br, We, be):
    raise NotImplementedError("write your pallas kernel here")


# speedup vs baseline: 1.2592x; 1.2592x over previous
import jax
import jax.numpy as jnp
from jax.experimental import pallas as pl
from jax.experimental.pallas import tpu as pltpu

_EP = 128


def _moe_body(r_ref, x_ref, wr_ref, br_ref, we_ref, be_ref, out_ref):
    n_exp = we_ref.shape[0]
    logits = jnp.dot(r_ref[...], wr_ref[...], preferred_element_type=jnp.float32)
    logits = logits + br_ref[...]
    m = jnp.max(logits, axis=-1, keepdims=True)
    u = jnp.exp(logits - m)
    denom = jnp.sum(u, axis=-1, keepdims=True)

    x = x_ref[...]
    acc = jnp.dot(u, be_ref[...], preferred_element_type=jnp.float32)
    for e in range(n_exp):
        y = jnp.dot(x, we_ref[e], preferred_element_type=jnp.float32)
        acc = acc + u[:, e : e + 1] * y
    out_ref[...] = acc / denom


def kernel(router_input, x, Wr, br, We, be):
    n, d = x.shape
    n_exp = We.shape[0]
    bn = 512

    xb = x.astype(jnp.bfloat16)
    web = We.astype(jnp.bfloat16)
    wrp = jnp.zeros((d, _EP), jnp.float32).at[:, :n_exp].set(Wr)
    brp = jnp.full((1, _EP), -jnp.inf, jnp.float32).at[0, :n_exp].set(br)
    bep = jnp.zeros((_EP, d), jnp.float32).at[:n_exp].set(be)

    return pl.pallas_call(
        _moe_body,
        grid=(n // bn,),
        in_specs=[
            pl.BlockSpec((bn, d), lambda i: (i, 0)),
            pl.BlockSpec((bn, d), lambda i: (i, 0)),
            pl.BlockSpec((d, _EP), lambda i: (0, 0)),
            pl.BlockSpec((1, _EP), lambda i: (0, 0)),
            pl.BlockSpec((n_exp, d, d), lambda i: (0, 0, 0)),
            pl.BlockSpec((_EP, d), lambda i: (0, 0)),
        ],
        out_specs=pl.BlockSpec((bn, d), lambda i: (i, 0)),
        out_shape=jax.ShapeDtypeStruct((n, d), jnp.float32),
        compiler_params=pltpu.CompilerParams(
            dimension_semantics=("arbitrary",),
        ),
    )(router_input, xb, wrp, brp, web, bep)
